# baseline (speedup 1.0000x reference)
import functools
import math

import jax
import jax.numpy as jnp
from jax import lax
from jax.experimental import pallas as pl
from jax.experimental.pallas import tpu as pltpu
from jax.experimental.pallas import tpu_sc as plsc

D = 64
K = 20
MARGIN = 0.5
NC = 2
NS = 16
NW = NC * NS
CHUNK = 32
NEG_CHUNK = CHUNK * K


@functools.lru_cache(maxsize=None)
def _make_sc_kernel(batch: int, n_items: int):
    bpw = batch // NW
    nchunks = bpw // CHUNK
    mesh = plsc.VectorSubcoreMesh(core_axis_name="c", subcore_axis_name="s")
    logvals = [math.log(c * n_items / K + 1.0) for c in range(K + 1)]

    @functools.partial(
        pl.kernel,
        mesh=mesh,
        compiler_params=pltpu.CompilerParams(
            needs_layout_passes=False, use_tc_tiling_on_sc=False),
        out_type=jax.ShapeDtypeStruct((NW, 16), jnp.float32),
        scratch_types=[
            pltpu.VMEM((bpw,), jnp.int32),
            pltpu.VMEM((bpw,), jnp.int32),
            pltpu.VMEM((bpw * K,), jnp.int32),
            pltpu.VMEM((CHUNK, D), jnp.float32),
            pltpu.VMEM((CHUNK, D), jnp.float32),
            pltpu.VMEM((NEG_CHUNK, D), jnp.float32),
            pltpu.VMEM((16,), jnp.float32),
            pltpu.SMEM((32,), jnp.float32),
            pltpu.SemaphoreType.DMA,
        ],
    )
    def sc(uid_hbm, pid_hbm, nid_hbm, uemb_hbm, iemb_hbm, out_hbm,
           uid_v, pid_v, nid_v, u_v, p_v, n_v, o_v, logtab, sem):
        wid = lax.axis_index("s") * NC + lax.axis_index("c")
        base = wid * bpw
        pltpu.sync_copy(uid_hbm.at[pl.ds(base, bpw)], uid_v)
        pltpu.sync_copy(pid_hbm.at[pl.ds(base, bpw)], pid_v)
        pltpu.sync_copy(nid_hbm.at[pl.ds(base * K, bpw * K)], nid_v)
        for c in range(K + 1):
            logtab[c] = jnp.float32(logvals[c])

        def chunk_body(ci, loss):
            off = ci * CHUNK
            cp_u = pltpu.async_copy(uemb_hbm.at[uid_v.at[pl.ds(off, CHUNK)]],
                                    u_v, sem)
            cp_p = pltpu.async_copy(iemb_hbm.at[pid_v.at[pl.ds(off, CHUNK)]],
                                    p_v, sem)
            cps = []
            for j in range(NEG_CHUNK // 128):
                cps.append(pltpu.async_copy(
                    iemb_hbm.at[nid_v.at[pl.ds(off * K + j * 128, 128)]],
                    n_v.at[pl.ds(j * 128, 128)], sem))
            cp_u.wait()
            cp_p.wait()
            for cp in cps:
                cp.wait()

            def pair_body(b, l):
                uv = [u_v[b, pl.ds(16 * c, 16)] for c in range(4)]
                pv = [p_v[b, pl.ds(16 * c, 16)] for c in range(4)]
                dp0 = uv[0] - pv[0]
                dp1 = uv[1] - pv[1]
                dp2 = uv[2] - pv[2]
                dp3 = uv[3] - pv[3]
                pos_dist = jnp.sum(dp0 * dp0 + dp1 * dp1
                                   + dp2 * dp2 + dp3 * dp3)
                thr = pos_dist + MARGIN
                nds = []
                for k in range(K):
                    r = b * K + k
                    d0 = uv[0] - n_v[r, pl.ds(0, 16)]
                    d1 = uv[1] - n_v[r, pl.ds(16, 16)]
                    d2 = uv[2] - n_v[r, pl.ds(32, 16)]
                    d3 = uv[3] - n_v[r, pl.ds(48, 16)]
                    nds.append(jnp.sum(d0 * d0 + d1 * d1 + d2 * d2 + d3 * d3))
                closest = functools.reduce(jnp.minimum, nds)
                cnt = jnp.int32(0)
                for nd in nds:
                    cnt = cnt + (thr > nd).astype(jnp.int32)
                lp = jnp.maximum(thr - closest, jnp.float32(0.0))
                return l + lp * logtab[cnt]

            return lax.fori_loop(0, CHUNK, pair_body, loss)

        loss = lax.fori_loop(0, nchunks, chunk_body, jnp.float32(0.0))
        o_v[...] = jnp.broadcast_to(loss, (16,))
        pltpu.sync_copy(o_v, out_hbm.at[wid])

    return sc


def kernel(user_ids, pos_item_ids, neg_item_ids, user_emb, item_emb):
    batch = user_ids.shape[0]
    n_items = item_emb.shape[0]
    sc = _make_sc_kernel(batch, n_items)
    partial = sc(user_ids, pos_item_ids, neg_item_ids.reshape(-1),
                 user_emb, item_emb)
    return partial[:, 0].sum()


# speedup vs baseline: 1.0033x; 1.0033x over previous
import functools
import math

import jax
import jax.numpy as jnp
from jax import lax
from jax.experimental import pallas as pl
from jax.experimental.pallas import tpu as pltpu
from jax.experimental.pallas import tpu_sc as plsc

D = 64
K = 20
MARGIN = 0.5
NC = 2
NS = 16
NW = NC * NS
CHUNK = 32
NEG_CHUNK = CHUNK * K


@functools.lru_cache(maxsize=None)
def _make_sc_kernel(batch: int, n_items: int):
    bpw = batch // NW
    nchunks = bpw // CHUNK
    mesh = plsc.VectorSubcoreMesh(core_axis_name="c", subcore_axis_name="s")
    logvals = [math.log(c * n_items / K + 1.0) for c in range(K + 1)]

    @functools.partial(
        pl.kernel,
        mesh=mesh,
        compiler_params=pltpu.CompilerParams(
            needs_layout_passes=False, use_tc_tiling_on_sc=False),
        out_type=jax.ShapeDtypeStruct((NW, 16), jnp.float32),
        scratch_types=[
            pltpu.VMEM((bpw,), jnp.int32),
            pltpu.VMEM((bpw,), jnp.int32),
            pltpu.VMEM((K, bpw), jnp.int32),
            pltpu.VMEM((CHUNK, D), jnp.float32),
            pltpu.VMEM((CHUNK, D), jnp.float32),
            pltpu.VMEM((K, CHUNK, D), jnp.float32),
            pltpu.VMEM((16,), jnp.float32),
            pltpu.SMEM((32,), jnp.float32),
            pltpu.SemaphoreType.DMA,
        ],
    )
    def sc(uid_hbm, pid_hbm, nid_hbm, uemb_hbm, iemb_hbm, out_hbm,
           uid_v, pid_v, nid_v, u_v, p_v, n_v, o_v, logtab, sem):
        wid = lax.axis_index("s") * NC + lax.axis_index("c")
        base = wid * bpw
        pltpu.sync_copy(uid_hbm.at[pl.ds(base, bpw)], uid_v)
        pltpu.sync_copy(pid_hbm.at[pl.ds(base, bpw)], pid_v)
        pltpu.sync_copy(nid_hbm.at[:, pl.ds(base, bpw)], nid_v)
        for c in range(K + 1):
            logtab[c] = jnp.float32(logvals[c])

        def chunk_body(ci, loss):
            off = ci * CHUNK
            cp_u = pltpu.async_copy(uemb_hbm.at[uid_v.at[pl.ds(off, CHUNK)]],
                                    u_v, sem)
            cp_p = pltpu.async_copy(iemb_hbm.at[pid_v.at[pl.ds(off, CHUNK)]],
                                    p_v, sem)
            cps = []
            for k in range(K):
                cps.append(pltpu.async_copy(
                    iemb_hbm.at[nid_v.at[k, pl.ds(off, CHUNK)]],
                    n_v.at[k], sem))
            cp_u.wait()
            cp_p.wait()
            for cp in cps:
                cp.wait()

            def pair_body(b, l):
                uv = [u_v[b, pl.ds(16 * c, 16)] for c in range(4)]
                pv = [p_v[b, pl.ds(16 * c, 16)] for c in range(4)]
                dp0 = uv[0] - pv[0]
                dp1 = uv[1] - pv[1]
                dp2 = uv[2] - pv[2]
                dp3 = uv[3] - pv[3]
                pos_dist = jnp.sum(dp0 * dp0 + dp1 * dp1
                                   + dp2 * dp2 + dp3 * dp3)
                thr = pos_dist + MARGIN
                nds = []
                for k in range(K):
                    d0 = uv[0] - n_v[k, b, pl.ds(0, 16)]
                    d1 = uv[1] - n_v[k, b, pl.ds(16, 16)]
                    d2 = uv[2] - n_v[k, b, pl.ds(32, 16)]
                    d3 = uv[3] - n_v[k, b, pl.ds(48, 16)]
                    nds.append(jnp.sum(d0 * d0 + d1 * d1 + d2 * d2 + d3 * d3))
                closest = functools.reduce(jnp.minimum, nds)
                cnt = jnp.int32(0)
                for nd in nds:
                    cnt = cnt + (thr > nd).astype(jnp.int32)
                lp = jnp.maximum(thr - closest, jnp.float32(0.0))
                return l + lp * logtab[cnt]

            return lax.fori_loop(0, CHUNK, pair_body, loss)

        loss = lax.fori_loop(0, nchunks, chunk_body, jnp.float32(0.0))
        o_v[...] = jnp.broadcast_to(loss, (16,))
        pltpu.sync_copy(o_v, out_hbm.at[wid])

    return sc


def kernel(user_ids, pos_item_ids, neg_item_ids, user_emb, item_emb):
    batch = user_ids.shape[0]
    n_items = item_emb.shape[0]
    sc = _make_sc_kernel(batch, n_items)
    partial = sc(user_ids, pos_item_ids, neg_item_ids.T,
                 user_emb, item_emb)
    return partial[:, 0].sum()
